# baseline (speedup 1.0000x reference)
import functools

import jax
import jax.numpy as jnp
from jax import lax
from jax.experimental import pallas as pl
from jax.experimental.pallas import tpu as pltpu
from jax.experimental.pallas import tpu_sc as plsc

N = 10000
E = 320000
D = 128
HID = 9
G = 16
NCLS = 10

NP = 10048
EP = 327680
EPT = EP // 32
EBLK = 128
AW = 16
NBLOCKS = EPT // EBLK
TW = 11
RB = 1256
NBLK = NP // RB

_PREC = jax.lax.Precision.HIGHEST


def _pack0_body(x_ref, w_ref, as_ref, ad_ref, out_ref):
    w = w_ref[...]
    wa = jnp.concatenate(
        [w,
         jnp.dot(w, as_ref[...], precision=_PREC),
         jnp.dot(w, ad_ref[...], precision=_PREC)], axis=1)
    out_ref[...] = jnp.dot(x_ref[...], wa, precision=_PREC,
                           preferred_element_type=jnp.float32)


def _pack0(xp, W0, a_s, a_d):
    return pl.pallas_call(
        _pack0_body,
        grid=(NBLK,),
        in_specs=[
            pl.BlockSpec((RB, D), lambda i: (i, 0)),
            pl.BlockSpec((D, HID), lambda i: (0, 0)),
            pl.BlockSpec((HID, 1), lambda i: (0, 0)),
            pl.BlockSpec((HID, 1), lambda i: (0, 0)),
        ],
        out_specs=pl.BlockSpec((RB, TW), lambda i: (i, 0)),
        out_shape=jax.ShapeDtypeStruct((NP, TW), jnp.float32),
    )(xp, W0, a_s, a_d)


_MESH = plsc.VectorSubcoreMesh(core_axis_name="c", subcore_axis_name="s")


@functools.partial(
    pl.kernel,
    out_type=jax.ShapeDtypeStruct((2 * NP, AW), jnp.float32),
    mesh=_MESH,
    compiler_params=pltpu.CompilerParams(needs_layout_passes=False,
                                         use_tc_tiling_on_sc=False),
    scratch_types=[
        pltpu.VMEM((NP * TW,), jnp.float32),
        pltpu.VMEM((EBLK, AW), jnp.float32),
        pltpu.VMEM((EBLK, AW), jnp.float32),
        pltpu.VMEM((EBLK,), jnp.int32),
        pltpu.VMEM((EBLK,), jnp.int32),
        pltpu.VMEM((1, 128), jnp.int32),
        pltpu.VMEM((1, 128), jnp.int32),
        pltpu.VMEM((1, 128), jnp.int32),
        pltpu.VMEM((1, 128), jnp.int32),
        pltpu.VMEM_SHARED((NP, AW), jnp.float32),
        pltpu.SemaphoreType.DMA,
        pltpu.SemaphoreType.DMA,
        pltpu.SemaphoreType.DMA,
        pltpu.SemaphoreType.DMA,
    ],
)
def _edge_pass(table_hbm, edges_hbm, out_hbm,
               table_v, ws_a, ws_b, sr_a, sr_b, ds_a, ds_b, dc_a, dc_b,
               acc, es_a, es_b, ss_a, ss_b):
    c = lax.axis_index("c")
    s = lax.axis_index("s")
    wid = c * 16 + s
    rows0 = s * (NP // 16)

    WS = (ws_a, ws_b)
    SR = (sr_a, sr_b)
    DS = (ds_a, ds_b)
    DC = (dc_a, dc_b)
    ES = (es_a, es_b)
    SS = (ss_a, ss_b)

    pltpu.sync_copy(table_hbm, table_v)
    iota = lax.iota(jnp.int32, 16)
    zv = jnp.zeros((16,), jnp.float32)
    for g in range(EBLK // 16):
        for t in range(AW):
            plsc.store_scatter(
                ws_a, [iota + g * 16, jnp.full((16,), t, jnp.int32)], zv)
    for r in range(0, NP // 16, EBLK):
        n = min(EBLK, NP // 16 - r)
        pltpu.sync_copy(ws_a.at[pl.ds(0, n)],
                        acc.at[pl.ds(rows0 + r, n)])
    plsc.subcore_barrier()

    def _edge_descs(b, k):
        base = wid * EPT + b * EBLK
        return (
            pltpu.make_async_copy(edges_hbm.at[0, pl.ds(base, EBLK)],
                                  SR[k], ES[k]),
            pltpu.make_async_copy(edges_hbm.at[1, pl.ds(base, 128)],
                                  DS[k].at[0], ES[k]),
        )

    def e_start(b, k):
        for d in _edge_descs(b, k):
            d.start()

    def e_wait(b, k):
        for d in _edge_descs(b, k):
            d.wait()

    def s_start(k):
        pltpu.async_copy(WS[k], acc.at[DC[k].at[0]], SS[k], add=True)

    def s_wait(k):
        pltpu.make_async_copy(WS[k], acc.at[DC[k].at[0]], SS[k]).wait()

    def compute(k):
        for j in range(EBLK // 16):
            sbase = SR[k][pl.ds(j * 16, 16)] * TW
            dl = DS[k][0, pl.ds(j * 16, 16)]
            DC[k][0, pl.ds(j * 16, 16)] = dl
            a1 = plsc.load_gather(table_v, [sbase + 9])
            a2 = plsc.load_gather(table_v, [dl * TW + 10])
            e = a1 + a2
            e = jnp.maximum(e, e * 0.2)
            ex = jnp.exp(e)
            rows = iota + (j * 16)
            plsc.store_scatter(WS[k], [rows, jnp.full((16,), 9, jnp.int32)],
                               ex)
            for t in range(HID):
                hv = plsc.load_gather(table_v, [sbase + t])
                plsc.store_scatter(
                    WS[k], [rows, jnp.full((16,), t, jnp.int32)], hv * ex)

    e_start(0, 0)

    def outer(i, carry):
        for kk in range(2):
            b = i * 2 + kk
            e_wait(b, kk)

            @pl.when(b + 1 < NBLOCKS)
            def _():
                e_start(b + 1, 1 - kk)

            @pl.when(b >= 2)
            def _():
                s_wait(kk)

            compute(kk)
            s_start(kk)
        return carry

    lax.fori_loop(0, NBLOCKS // 2, outer, 0)
    s_wait(0)
    s_wait(1)
    plsc.subcore_barrier()
    pltpu.sync_copy(acc.at[pl.ds(rows0, NP // 16)],
                    out_hbm.at[pl.ds(c * NP + rows0, NP // 16)])


def _combine_body(p_ref, b_ref, w_ref, as_ref, ad_ref, o_ref, t_ref):
    p = p_ref[...]
    num = p[0, :, 0:HID] + p[1, :, 0:HID]
    den = p[0, :, HID:HID + 1] + p[1, :, HID:HID + 1]
    o = jnp.maximum(num / (den + 1e-16) + b_ref[...], 0.0)
    o_ref[...] = o
    w = w_ref[...]
    wa = jnp.concatenate(
        [w,
         jnp.dot(w, as_ref[...], precision=_PREC),
         jnp.dot(w, ad_ref[...], precision=_PREC)], axis=1)
    t_ref[...] = jnp.dot(o, wa, precision=_PREC,
                         preferred_element_type=jnp.float32)


def _combine(parts, b, W, a_s, a_d):
    return pl.pallas_call(
        _combine_body,
        grid=(NBLK,),
        in_specs=[
            pl.BlockSpec((2, RB, AW), lambda i: (0, i, 0)),
            pl.BlockSpec((1, HID), lambda i: (0, 0)),
            pl.BlockSpec((HID, HID), lambda i: (0, 0)),
            pl.BlockSpec((HID, 1), lambda i: (0, 0)),
            pl.BlockSpec((HID, 1), lambda i: (0, 0)),
        ],
        out_specs=[
            pl.BlockSpec((RB, HID), lambda i: (i, 0)),
            pl.BlockSpec((RB, TW), lambda i: (i, 0)),
        ],
        out_shape=[
            jax.ShapeDtypeStruct((NP, HID), jnp.float32),
            jax.ShapeDtypeStruct((NP, TW), jnp.float32),
        ],
    )(parts, b, W, a_s, a_d)


def _final_body(o0_ref, o1_ref, p_ref, b_ref, batch_ref, fcw_ref, fcb_ref,
                out_ref, pool_ref):
    i = pl.program_id(0)

    @pl.when(i == 0)
    def _init():
        pool_ref[...] = jnp.full((G, 32), -jnp.inf, jnp.float32)

    p = p_ref[...]
    num = p[0, :, 0:HID] + p[1, :, 0:HID]
    den = p[0, :, HID:HID + 1] + p[1, :, HID:HID + 1]
    o2 = jnp.maximum(num / (den + 1e-16) + b_ref[...], 0.0)
    jk = jnp.concatenate(
        [o0_ref[...], o1_ref[...], o2,
         jnp.full((RB, 32 - 3 * HID), -jnp.inf, jnp.float32)], axis=1)
    bb = batch_ref[...]
    for g in range(G):
        vals = jnp.where(bb == g, jk, -jnp.inf)
        mx = jnp.max(vals, axis=0, keepdims=True)
        pool_ref[pl.ds(g, 1), :] = jnp.maximum(pool_ref[pl.ds(g, 1), :], mx)

    @pl.when(i == NBLK - 1)
    def _fc():
        pooled = pool_ref[...][:, 0:3 * HID]
        pooled = jnp.where(jnp.isfinite(pooled), pooled, 0.0)
        out_ref[...] = jnp.dot(pooled, fcw_ref[...], precision=_PREC,
                               preferred_element_type=jnp.float32) + fcb_ref[...]


def _final(o0, o1, parts, b, batch_p, fc_W, fc_b):
    return pl.pallas_call(
        _final_body,
        grid=(NBLK,),
        in_specs=[
            pl.BlockSpec((RB, HID), lambda i: (i, 0)),
            pl.BlockSpec((RB, HID), lambda i: (i, 0)),
            pl.BlockSpec((2, RB, AW), lambda i: (0, i, 0)),
            pl.BlockSpec((1, HID), lambda i: (0, 0)),
            pl.BlockSpec((RB, 1), lambda i: (i, 0)),
            pl.BlockSpec((3 * HID, NCLS), lambda i: (0, 0)),
            pl.BlockSpec((1, NCLS), lambda i: (0, 0)),
        ],
        out_specs=pl.BlockSpec((G, NCLS), lambda i: (0, 0)),
        out_shape=jax.ShapeDtypeStruct((G, NCLS), jnp.float32),
        scratch_shapes=[pltpu.VMEM((G, 32), jnp.float32)],
    )(o0, o1, parts, b, batch_p, fc_W, fc_b)


def kernel(x, edge_index, batch, W0, a_src0, a_dst0, b0,
           W1, a_src1, a_dst1, b1, W2, a_src2, a_dst2, b2, fc_W, fc_b):
    xp = jnp.pad(x, ((0, NP - N), (0, 0)))
    ep = jnp.pad(edge_index, ((0, 0), (0, EP - E)), constant_values=N)
    bp = jnp.pad(batch.astype(jnp.int32), (0, NP - N),
                 constant_values=G).reshape(NP, 1)

    table0 = _pack0(xp, W0, a_src0.reshape(HID, 1), a_dst0.reshape(HID, 1))
    part0 = _edge_pass(table0.reshape(NP * TW), ep).reshape(2, NP, AW)
    o0, table1 = _combine(part0, b0.reshape(1, HID), W1,
                          a_src1.reshape(HID, 1), a_dst1.reshape(HID, 1))
    part1 = _edge_pass(table1.reshape(NP * TW), ep).reshape(2, NP, AW)
    o1, table2 = _combine(part1, b1.reshape(1, HID), W2,
                          a_src2.reshape(HID, 1), a_dst2.reshape(HID, 1))
    part2 = _edge_pass(table2.reshape(NP * TW), ep).reshape(2, NP, AW)
    return _final(o0, o1, part2, b2.reshape(1, HID), bp, fc_W,
                  fc_b.reshape(1, NCLS))


# speedup vs baseline: 1.0080x; 1.0080x over previous
import functools

import jax
import jax.numpy as jnp
from jax import lax
from jax.experimental import pallas as pl
from jax.experimental.pallas import tpu as pltpu
from jax.experimental.pallas import tpu_sc as plsc

N = 10000
E = 320000
D = 128
HID = 9
G = 16
NCLS = 10

NP = 10048
EP = 327680
EPT = EP // 32
EBLK = 128
AW = 16
NBLOCKS = EPT // EBLK
TW = 11
RB = 1256
NBLK = NP // RB

_PREC = jax.lax.Precision.HIGHEST


def _pack0_body(x_ref, w_ref, as_ref, ad_ref, out_ref):
    w = w_ref[...]
    wa = jnp.concatenate(
        [w,
         jnp.dot(w, as_ref[...], precision=_PREC),
         jnp.dot(w, ad_ref[...], precision=_PREC)], axis=1)
    out_ref[...] = jnp.dot(x_ref[...], wa, precision=_PREC,
                           preferred_element_type=jnp.float32)


def _pack0(xp, W0, a_s, a_d):
    return pl.pallas_call(
        _pack0_body,
        grid=(NBLK,),
        in_specs=[
            pl.BlockSpec((RB, D), lambda i: (i, 0)),
            pl.BlockSpec((D, HID), lambda i: (0, 0)),
            pl.BlockSpec((HID, 1), lambda i: (0, 0)),
            pl.BlockSpec((HID, 1), lambda i: (0, 0)),
        ],
        out_specs=pl.BlockSpec((RB, TW), lambda i: (i, 0)),
        out_shape=jax.ShapeDtypeStruct((NP, TW), jnp.float32),
    )(xp, W0, a_s, a_d)


_MESH = plsc.VectorSubcoreMesh(core_axis_name="c", subcore_axis_name="s")


@functools.partial(
    pl.kernel,
    out_type=jax.ShapeDtypeStruct((2 * NP, AW), jnp.float32),
    mesh=_MESH,
    compiler_params=pltpu.CompilerParams(needs_layout_passes=False,
                                         use_tc_tiling_on_sc=False),
    scratch_types=[
        pltpu.VMEM((NP * TW,), jnp.float32),
        pltpu.VMEM((EBLK, AW), jnp.float32),
        pltpu.VMEM((EBLK, AW), jnp.float32),
        pltpu.VMEM((EBLK,), jnp.int32),
        pltpu.VMEM((EBLK,), jnp.int32),
        pltpu.VMEM((1, 128), jnp.int32),
        pltpu.VMEM((1, 128), jnp.int32),
        pltpu.VMEM((1, 128), jnp.int32),
        pltpu.VMEM((1, 128), jnp.int32),
        pltpu.VMEM_SHARED((NP, AW), jnp.float32),
        pltpu.SemaphoreType.DMA,
        pltpu.SemaphoreType.DMA,
        pltpu.SemaphoreType.DMA,
        pltpu.SemaphoreType.DMA,
    ],
)
def _edge_pass(table_hbm, edges_hbm, out_hbm,
               table_v, ws_a, ws_b, sr_a, sr_b, ds_a, ds_b, dc_a, dc_b,
               acc, es_a, es_b, ss_a, ss_b):
    c = lax.axis_index("c")
    s = lax.axis_index("s")
    wid = c * 16 + s
    rows0 = s * (NP // 16)

    WS = (ws_a, ws_b)
    SR = (sr_a, sr_b)
    DS = (ds_a, ds_b)
    DC = (dc_a, dc_b)
    ES = (es_a, es_b)
    SS = (ss_a, ss_b)

    table_dma = pltpu.make_async_copy(table_hbm, table_v, es_b)
    table_dma.start()
    iota = lax.iota(jnp.int32, 16)
    zv = jnp.zeros((16,), jnp.float32)
    for g in range(EBLK // 16):
        for t in range(AW):
            plsc.store_scatter(
                ws_a, [iota + g * 16, jnp.full((16,), t, jnp.int32)], zv)
    for r in range(0, NP // 16, EBLK):
        n = min(EBLK, NP // 16 - r)
        pltpu.sync_copy(ws_a.at[pl.ds(0, n)],
                        acc.at[pl.ds(rows0 + r, n)])
    table_dma.wait()
    plsc.subcore_barrier()

    def _edge_descs(b, k):
        base = wid * EPT + b * EBLK
        return (
            pltpu.make_async_copy(edges_hbm.at[0, pl.ds(base, EBLK)],
                                  SR[k], ES[k]),
            pltpu.make_async_copy(edges_hbm.at[1, pl.ds(base, 128)],
                                  DS[k].at[0], ES[k]),
        )

    def e_start(b, k):
        for d in _edge_descs(b, k):
            d.start()

    def e_wait(b, k):
        for d in _edge_descs(b, k):
            d.wait()

    def s_start(k):
        pltpu.async_copy(WS[k], acc.at[DC[k].at[0]], SS[k], add=True)

    def s_wait(k):
        pltpu.make_async_copy(WS[k], acc.at[DC[k].at[0]], SS[k]).wait()

    def compute(k):
        for j in range(EBLK // 16):
            sbase = SR[k][pl.ds(j * 16, 16)] * TW
            dl = DS[k][0, pl.ds(j * 16, 16)]
            DC[k][0, pl.ds(j * 16, 16)] = dl
            a1 = plsc.load_gather(table_v, [sbase + 9])
            a2 = plsc.load_gather(table_v, [dl * TW + 10])
            e = a1 + a2
            e = jnp.maximum(e, e * 0.2)
            ex = jnp.exp(e)
            rows = iota + (j * 16)
            plsc.store_scatter(WS[k], [rows, jnp.full((16,), 9, jnp.int32)],
                               ex)
            for t in range(HID):
                hv = plsc.load_gather(table_v, [sbase + t])
                plsc.store_scatter(
                    WS[k], [rows, jnp.full((16,), t, jnp.int32)], hv * ex)

    e_start(0, 0)

    def outer(i, carry):
        for kk in range(2):
            b = i * 2 + kk
            e_wait(b, kk)

            @pl.when(b + 1 < NBLOCKS)
            def _():
                e_start(b + 1, 1 - kk)

            @pl.when(b >= 2)
            def _():
                s_wait(kk)

            compute(kk)
            s_start(kk)
        return carry

    lax.fori_loop(0, NBLOCKS // 2, outer, 0)
    s_wait(0)
    s_wait(1)
    plsc.subcore_barrier()
    pltpu.sync_copy(acc.at[pl.ds(rows0, NP // 16)],
                    out_hbm.at[pl.ds(c * NP + rows0, NP // 16)])


def _combine_body(p_ref, b_ref, w_ref, as_ref, ad_ref, o_ref, t_ref):
    p = p_ref[...]
    num = p[0, :, 0:HID] + p[1, :, 0:HID]
    den = p[0, :, HID:HID + 1] + p[1, :, HID:HID + 1]
    o = jnp.maximum(num / (den + 1e-16) + b_ref[...], 0.0)
    o_ref[...] = o
    w = w_ref[...]
    wa = jnp.concatenate(
        [w,
         jnp.dot(w, as_ref[...], precision=_PREC),
         jnp.dot(w, ad_ref[...], precision=_PREC)], axis=1)
    t_ref[...] = jnp.dot(o, wa, precision=_PREC,
                         preferred_element_type=jnp.float32)


def _combine(parts, b, W, a_s, a_d):
    return pl.pallas_call(
        _combine_body,
        grid=(NBLK,),
        in_specs=[
            pl.BlockSpec((2, RB, AW), lambda i: (0, i, 0)),
            pl.BlockSpec((1, HID), lambda i: (0, 0)),
            pl.BlockSpec((HID, HID), lambda i: (0, 0)),
            pl.BlockSpec((HID, 1), lambda i: (0, 0)),
            pl.BlockSpec((HID, 1), lambda i: (0, 0)),
        ],
        out_specs=[
            pl.BlockSpec((RB, HID), lambda i: (i, 0)),
            pl.BlockSpec((RB, TW), lambda i: (i, 0)),
        ],
        out_shape=[
            jax.ShapeDtypeStruct((NP, HID), jnp.float32),
            jax.ShapeDtypeStruct((NP, TW), jnp.float32),
        ],
    )(parts, b, W, a_s, a_d)


def _final_body(o0_ref, o1_ref, p_ref, b_ref, batch_ref, fcw_ref, fcb_ref,
                out_ref, pool_ref):
    i = pl.program_id(0)

    @pl.when(i == 0)
    def _init():
        pool_ref[...] = jnp.full((G, 32), -jnp.inf, jnp.float32)

    p = p_ref[...]
    num = p[0, :, 0:HID] + p[1, :, 0:HID]
    den = p[0, :, HID:HID + 1] + p[1, :, HID:HID + 1]
    o2 = jnp.maximum(num / (den + 1e-16) + b_ref[...], 0.0)
    jk = jnp.concatenate(
        [o0_ref[...], o1_ref[...], o2,
         jnp.full((RB, 32 - 3 * HID), -jnp.inf, jnp.float32)], axis=1)
    bb = batch_ref[...]
    for g in range(G):
        vals = jnp.where(bb == g, jk, -jnp.inf)
        mx = jnp.max(vals, axis=0, keepdims=True)
        pool_ref[pl.ds(g, 1), :] = jnp.maximum(pool_ref[pl.ds(g, 1), :], mx)

    @pl.when(i == NBLK - 1)
    def _fc():
        pooled = pool_ref[...][:, 0:3 * HID]
        pooled = jnp.where(jnp.isfinite(pooled), pooled, 0.0)
        out_ref[...] = jnp.dot(pooled, fcw_ref[...], precision=_PREC,
                               preferred_element_type=jnp.float32) + fcb_ref[...]


def _final(o0, o1, parts, b, batch_p, fc_W, fc_b):
    return pl.pallas_call(
        _final_body,
        grid=(NBLK,),
        in_specs=[
            pl.BlockSpec((RB, HID), lambda i: (i, 0)),
            pl.BlockSpec((RB, HID), lambda i: (i, 0)),
            pl.BlockSpec((2, RB, AW), lambda i: (0, i, 0)),
            pl.BlockSpec((1, HID), lambda i: (0, 0)),
            pl.BlockSpec((RB, 1), lambda i: (i, 0)),
            pl.BlockSpec((3 * HID, NCLS), lambda i: (0, 0)),
            pl.BlockSpec((1, NCLS), lambda i: (0, 0)),
        ],
        out_specs=pl.BlockSpec((G, NCLS), lambda i: (0, 0)),
        out_shape=jax.ShapeDtypeStruct((G, NCLS), jnp.float32),
        scratch_shapes=[pltpu.VMEM((G, 32), jnp.float32)],
    )(o0, o1, parts, b, batch_p, fc_W, fc_b)


def kernel(x, edge_index, batch, W0, a_src0, a_dst0, b0,
           W1, a_src1, a_dst1, b1, W2, a_src2, a_dst2, b2, fc_W, fc_b):
    xp = jnp.pad(x, ((0, NP - N), (0, 0)))
    ep = jnp.pad(edge_index, ((0, 0), (0, EP - E)), constant_values=N)
    bp = jnp.pad(batch.astype(jnp.int32), (0, NP - N),
                 constant_values=G).reshape(NP, 1)

    table0 = _pack0(xp, W0, a_src0.reshape(HID, 1), a_dst0.reshape(HID, 1))
    part0 = _edge_pass(table0.reshape(NP * TW), ep).reshape(2, NP, AW)
    o0, table1 = _combine(part0, b0.reshape(1, HID), W1,
                          a_src1.reshape(HID, 1), a_dst1.reshape(HID, 1))
    part1 = _edge_pass(table1.reshape(NP * TW), ep).reshape(2, NP, AW)
    o1, table2 = _combine(part1, b1.reshape(1, HID), W2,
                          a_src2.reshape(HID, 1), a_dst2.reshape(HID, 1))
    part2 = _edge_pass(table2.reshape(NP * TW), ep).reshape(2, NP, AW)
    return _final(o0, o1, part2, b2.reshape(1, HID), bp, fc_W,
                  fc_b.reshape(1, NCLS))
